# baseline (speedup 1.0000x reference)
import jax
import jax.numpy as jnp
from jax.experimental import pallas as pl
from jax.experimental.pallas import tpu as pltpu

_BR = 2000


def _fused_mlp_kernel(x_ref, w1_ref, b1_ref, w2_ref, b2_ref, o_ref):
    h = jax.lax.dot_general(
        x_ref[...], w1_ref[...],
        dimension_numbers=(((1,), (0,)), ((), ())),
        preferred_element_type=jnp.float32,
    )
    h = jnp.maximum(h + b1_ref[...], 0.0)
    o = jax.lax.dot_general(
        h, w2_ref[...],
        dimension_numbers=(((1,), (0,)), ((), ())),
        preferred_element_type=jnp.float32,
    )
    o_ref[...] = o + b2_ref[...]


def kernel(x_transaction, x_merchant, x_category, edge_index_belongs_to, edge_index_has_category, W_enc_t, b_enc_t, W_enc_m, b_enc_m, W_enc_c, b_enc_c, lin_l_bm_0, bias_bm_0, lin_r_bm_0, lin_l_tc_0, bias_tc_0, lin_r_tc_0, lin_l_bm_1, bias_bm_1, lin_r_bm_1, lin_l_tc_1, bias_tc_1, lin_r_tc_1, W_cls, b_cls):
    NT, D = x_transaction.shape
    H = W_enc_t.shape[1]
    OUT = W_cls.shape[1]
    OUTP = ((OUT + 511) // 512) * 512

    grid = (NT // _BR,)

    b1 = b_enc_t.reshape(1, H)
    W2 = jnp.pad(W_cls, ((0, 0), (0, OUTP - OUT)))
    b2 = jnp.pad(b_cls, (0, OUTP - OUT)).reshape(1, OUTP)

    return pl.pallas_call(
        _fused_mlp_kernel,
        grid=grid,
        in_specs=[
            pl.BlockSpec((_BR, D), lambda i: (i, 0)),
            pl.BlockSpec((D, H), lambda i: (0, 0)),
            pl.BlockSpec((1, H), lambda i: (0, 0)),
            pl.BlockSpec((H, OUTP), lambda i: (0, 0)),
            pl.BlockSpec((1, OUTP), lambda i: (0, 0)),
        ],
        out_specs=pl.BlockSpec((_BR, OUTP), lambda i: (i, 0)),
        out_shape=jax.ShapeDtypeStruct((NT, OUT), jnp.float32),
        compiler_params=pltpu.CompilerParams(
            dimension_semantics=("parallel",),
            disable_bounds_checks=True,
        ),
    )(x_transaction, W_enc_t, b1, W2, b2)
